# Initial kernel scaffold; baseline (speedup 1.0000x reference)
import jax
import jax.numpy as jnp
from jax.experimental import pallas as pl


def kernel(tables, W1, b1, W2, b2, W3, b3):
    raise NotImplementedError("write your pallas kernel here")


# speedup vs baseline: 30.4335x; 30.4335x over previous
import functools

import numpy as np
import jax
import jax.numpy as jnp
from jax import lax
from jax.experimental import pallas as pl
from jax.experimental.pallas import tpu as pltpu
from jax.experimental.pallas import tpu_sc as plsc

W_RES = 1024
H_RES = 1024
LVL = 16
F_DIM = 2
LOG2_T = 19
TBL = 2 ** LOG2_T
HASH_K = np.uint32(2654435761)
HMASK = np.uint32(TBL - 1)
HIDDEN = 64
N_PIX = W_RES * H_RES

ROWS_PER_TEC = H_RES // 32


def _ceil8(n):
    return (n + 7) // 8 * 8


@functools.lru_cache(maxsize=1)
def _host_consts():
    b = np.exp((np.log(2048.0) - np.log(16.0)) / (LVL - 1))
    res = np.floor(16.0 * (b ** np.arange(LVL))).astype(np.float32)
    norm = (np.arange(1024, dtype=np.float32) / np.float32(1024))

    cix = np.zeros((LVL, 1024), np.int32)
    fx = np.zeros((LVL, 1024), np.float32)
    gridw = []
    for l in range(LVL):
        r = np.float32(res[l])
        sx = (norm * r).astype(np.float32)
        px = np.floor(sx)
        ix = px.astype(np.int32)
        gridw.append(int(ix.max()) + 2)
        cix[l] = ix
        fx[l] = sx - px

    seg = [_ceil8(2 * gridw[l]) for l in range(LVL - 1)] + [_ceil8(1024)]
    off = np.cumsum([0] + seg).tolist()
    perrow = off[-1]

    fy = np.zeros((1024, LVL), np.float32)
    for l in range(LVL):
        r = np.float32(res[l])
        sy = (norm * r).astype(np.float32)
        py = np.floor(sy)
        fy[:, l] = sy - py
    return gridw, seg, off, perrow, res, fx.reshape(-1), fy.reshape(-1)


N_SLAB = 3


def _sc_encode(tflat, idxh, fyh, gridw, seg, res):
    info = plsc.get_sparse_core_info()
    nc = info.num_cores
    perrow = sum(seg)
    maxseg = max(seg)

    def body(t_hbm, idx_hbm, fy_hbm, enc_hbm,
             fyv, encv, idxvs, slabs, idx_sems, gat_sems):
        wid = lax.axis_index("s") * nc + lax.axis_index("c")
        row_base = wid * ROWS_PER_TEC

        pltpu.sync_copy(fy_hbm.at[pl.ds(row_base * LVL, ROWS_PER_TEC * LVL)], fyv)

        zero16 = jnp.zeros((16,), jnp.int32)
        one16 = jnp.ones((16,), jnp.int32)
        lane16 = jnp.arange(16, dtype=jnp.int32)

        def row_body(rl, carry):
            row = row_base + rl
            idx_copies = []
            for l in range(LVL):
                cp = pltpu.make_async_copy(
                    idx_hbm.at[pl.ds(row * perrow + sum(seg[:l]), seg[l])],
                    idxvs[l], idx_sems[l])
                cp.start()
                idx_copies.append(cp)

            def start_gather(l):
                idx_copies[l].wait()
                cp = pltpu.make_async_copy(
                    t_hbm.at[idxvs[l]],
                    slabs[l % N_SLAB].at[pl.ds(0, seg[l])],
                    gat_sems[l % N_SLAB])
                cp.start()
                return cp

            def compute(l, slab):
                if l == LVL - 1:
                    def cbody15(ci, c):
                        basec = ci * 16
                        vcol = lane16 + basec
                        f0 = plsc.load_gather(slab, [vcol, zero16])
                        f1 = plsc.load_gather(slab, [vcol, one16])
                        encv[2 * l, pl.ds(basec, 16)] = f0
                        encv[2 * l + 1, pl.ds(basec, 16)] = f1
                        return c
                    lax.fori_loop(0, 64, cbody15, 0, unroll=2)
                    return
                g = gridw[l]
                rinv = float(res[l]) / 1024.0
                vfy = plsc.load_gather(
                    fyv, [jnp.full((16,), rl * LVL + l, jnp.int32)])

                def cbody(ci, c, g=g, rinv=rinv, slab=slab, vfy=vfy, l=l):
                    basec = ci * 16
                    vcolf = (lane16 + basec).astype(jnp.float32)
                    vs = vcolf * jnp.float32(rinv)
                    vix = vs.astype(jnp.int32)
                    vfx = vs - vix.astype(jnp.float32)
                    vix1 = vix + 1
                    vixg = vix + g
                    vixg1 = vixg + 1
                    c00f0 = plsc.load_gather(slab, [vix, zero16])
                    c00f1 = plsc.load_gather(slab, [vix, one16])
                    c10f0 = plsc.load_gather(slab, [vix1, zero16])
                    c10f1 = plsc.load_gather(slab, [vix1, one16])
                    c01f0 = plsc.load_gather(slab, [vixg, zero16])
                    c01f1 = plsc.load_gather(slab, [vixg, one16])
                    c11f0 = plsc.load_gather(slab, [vixg1, zero16])
                    c11f1 = plsc.load_gather(slab, [vixg1, one16])
                    a0 = c00f0 + vfx * (c10f0 - c00f0)
                    a1 = c00f1 + vfx * (c10f1 - c00f1)
                    bb0 = c01f0 + vfx * (c11f0 - c01f0)
                    bb1 = c01f1 + vfx * (c11f1 - c01f1)
                    f0 = a0 + vfy * (bb0 - a0)
                    f1 = a1 + vfy * (bb1 - a1)
                    encv[2 * l, pl.ds(basec, 16)] = f0
                    encv[2 * l + 1, pl.ds(basec, 16)] = f1
                    return c
                lax.fori_loop(0, 64, cbody, 0, unroll=2)

            gat = {}
            for j in range(N_SLAB):
                gat[j] = start_gather(j)
            for l in range(LVL):
                gat[l].wait()
                compute(l, slabs[l % N_SLAB])
                if l + N_SLAB < LVL:
                    gat[l + N_SLAB] = start_gather(l + N_SLAB)
            pltpu.sync_copy(encv, enc_hbm.at[:, pl.ds(row * 1024, 1024)])
            return carry

        lax.fori_loop(0, ROWS_PER_TEC, row_body, 0)

    mesh = plsc.VectorSubcoreMesh(core_axis_name="c", subcore_axis_name="s")
    scratch = [
        pltpu.VMEM((ROWS_PER_TEC * LVL,), jnp.float32),
        pltpu.VMEM((2 * LVL, 1024), jnp.float32),
        [pltpu.VMEM((seg[l],), jnp.int32) for l in range(LVL)],
        [pltpu.VMEM((maxseg, F_DIM), jnp.float32) for _ in range(N_SLAB)],
        [pltpu.SemaphoreType.DMA for _ in range(LVL)],
        [pltpu.SemaphoreType.DMA for _ in range(N_SLAB)],
    ]
    k = pl.kernel(
        body,
        out_type=jax.ShapeDtypeStruct((2 * LVL, N_PIX), jnp.float32),
        mesh=mesh,
        scratch_types=scratch,
        compiler_params=pltpu.CompilerParams(use_tc_tiling_on_sc=False,
                                             needs_layout_passes=False),
    )
    return k(tflat, idxh, fyh)


def _mlp_body(e_ref, w1_ref, b1_ref, w2_ref, b2_ref, w3_ref, b3_ref, o_ref):
    e = e_ref[...]
    h = jnp.dot(w1_ref[...], e, preferred_element_type=jnp.float32) + b1_ref[...]
    h = jnp.maximum(h, 0.0)
    h = jnp.dot(w2_ref[...], h, preferred_element_type=jnp.float32) + b2_ref[...]
    h = jnp.maximum(h, 0.0)
    o = jnp.dot(w3_ref[...], h, preferred_element_type=jnp.float32) + b3_ref[...]
    o_ref[...] = jax.nn.sigmoid(o)


def _mlp(enc, w1t, b1, w2t, b2, w3t, b3):
    bn = 4096
    grid = (N_PIX // bn,)
    out = pl.pallas_call(
        _mlp_body,
        grid=grid,
        in_specs=[
            pl.BlockSpec((2 * LVL, bn), lambda i: (0, i)),
            pl.BlockSpec((HIDDEN, 2 * LVL), lambda i: (0, 0)),
            pl.BlockSpec((HIDDEN, 1), lambda i: (0, 0)),
            pl.BlockSpec((HIDDEN, HIDDEN), lambda i: (0, 0)),
            pl.BlockSpec((HIDDEN, 1), lambda i: (0, 0)),
            pl.BlockSpec((8, HIDDEN), lambda i: (0, 0)),
            pl.BlockSpec((8, 1), lambda i: (0, 0)),
        ],
        out_specs=pl.BlockSpec((8, bn), lambda i: (0, i)),
        out_shape=jax.ShapeDtypeStruct((8, N_PIX), jnp.float32),
    )(enc, w1t, b1, w2t, b2, w3t, b3)
    return out[:3]


def kernel(tables, W1, b1, W2, b2, W3, b3):
    gridw, seg_l, off_l, perrow, res, fx_np, fy_np = _host_consts()
    tflat = tables.reshape(LVL * TBL, F_DIM)
    idx_np = _host_idx()
    enc = _sc_encode(
        tflat,
        jnp.asarray(idx_np.reshape(-1)),
        jnp.asarray(fy_np),
        gridw, seg_l, res)
    w1t = W1.T
    w2t = W2.T
    w3t = jnp.zeros((8, HIDDEN), jnp.float32).at[:3].set(W3.T)
    b3p = jnp.zeros((8, 1), jnp.float32).at[:3, 0].set(b3)
    out = _mlp(enc, w1t, b1.reshape(HIDDEN, 1), w2t, b2.reshape(HIDDEN, 1),
               w3t, b3p)
    return out.reshape(3, H_RES, W_RES)[None]


@functools.lru_cache(maxsize=1)
def _host_idx():
    gridw, seg, off, perrow, _, _, _ = _host_consts()
    return _build_idx(tuple(gridw), tuple(seg), tuple(off), perrow)


def _build_idx(gridw, seg, off, perrow):
    b = np.exp((np.log(2048.0) - np.log(16.0)) / (LVL - 1))
    res = np.floor(16.0 * (b ** np.arange(LVL))).astype(np.float32)
    norm = (np.arange(1024, dtype=np.float32) / np.float32(1024))
    idxh = np.zeros((1024, perrow), np.int32)
    for l in range(LVL):
        r = np.float32(res[l])
        sy = (norm * r).astype(np.float32)
        py = np.floor(sy)
        iy = py.astype(np.uint32)
        hy0 = ((iy * HASH_K) & HMASK).astype(np.int64)
        hy1 = (((iy + np.uint32(1)) * HASH_K) & HMASK).astype(np.int64)
        base = l * TBL
        if l == LVL - 1:
            a = (np.arange(1024, dtype=np.int64) * 2)
            idxh[:, off[l]:off[l] + 1024] = (base + (a[None, :] ^ hy0[:, None])).astype(np.int32)
            idxh[:, off[l] + 1024:off[l + 1]] = base
        else:
            g = gridw[l]
            a = np.arange(g, dtype=np.int64)
            idxh[:, off[l]:off[l] + g] = (base + (a[None, :] ^ hy0[:, None])).astype(np.int32)
            idxh[:, off[l] + g:off[l] + 2 * g] = (base + (a[None, :] ^ hy1[:, None])).astype(np.int32)
            idxh[:, off[l] + 2 * g:off[l + 1]] = base
    return idxh


# speedup vs baseline: 119.4654x; 3.9255x over previous
import functools

import numpy as np
import jax
import jax.numpy as jnp
from jax import lax
from jax.experimental import pallas as pl
from jax.experimental.pallas import tpu as pltpu
from jax.experimental.pallas import tpu_sc as plsc

W_RES = 1024
H_RES = 1024
LVL = 16
F_DIM = 2
LOG2_T = 19
TBL = 2 ** LOG2_T
HASH_K = np.uint32(2654435761)
HMASK = np.uint32(TBL - 1)
HIDDEN = 64
N_PIX = W_RES * H_RES

ROWS_PER_TEC = H_RES // 32
NSPAN = TBL // 128
ROWS_PER_LVL = 2 * NSPAN


def _ceil8(n):
    return (n + 7) // 8 * 8


def _next_pow2(n):
    p = 1
    while p < n:
        p *= 2
    return p


@functools.lru_cache(maxsize=1)
def _host_consts():
    b = np.exp((np.log(2048.0) - np.log(16.0)) / (LVL - 1))
    res = np.floor(16.0 * (b ** np.arange(LVL))).astype(np.float32)
    norm = (np.arange(1024, dtype=np.float32) / np.float32(1024))

    gridw = []
    for l in range(LVL):
        r = np.float32(res[l])
        sx = (norm * r).astype(np.float32)
        ix = np.floor(sx).astype(np.int32)
        gridw.append(int(ix.max()) + 2)

    nb = [-(-gridw[l] // 128) for l in range(LVL)]
    nbp2 = [_next_pow2(nb[l]) for l in range(LVL)]
    seg = [_ceil8(4 * nbp2[l]) for l in range(LVL - 1)] + [2 * nbp2[LVL - 1]]
    off = np.cumsum([0] + seg).tolist()
    perrow = off[-1]

    fy = np.zeros((1024, LVL), np.float32)
    hc = np.zeros((1024, LVL, 4), np.int32)
    idxh = np.zeros((1024, perrow), np.int32)
    for l in range(LVL):
        r = np.float32(res[l])
        sy = (norm * r).astype(np.float32)
        py = np.floor(sy)
        iy = py.astype(np.uint32)
        fy[:, l] = sy - py
        hy0 = ((iy * HASH_K) & HMASK).astype(np.int64)
        hy1 = (((iy + np.uint32(1)) * HASH_K) & HMASK).astype(np.int64)
        base = l * ROWS_PER_LVL
        for rr in range(1024):
            h0hi, h0lo = int(hy0[rr]) >> 7, int(hy0[rr]) & 127
            h1hi, h1lo = int(hy1[rr]) >> 7, int(hy1[rr]) & 127
            if l == LVL - 1:
                for f in range(2):
                    for k in range(nbp2[l]):
                        idxh[rr, off[l] + f * nbp2[l] + k] = (
                            base + ((k ^ h0hi) << 1) + f) if k < nb[l] else base
                hc[rr, l, 0] = h0lo
                hc[rr, l, 1] = nbp2[l] * 128 + h0lo
            else:
                for p in range(4):
                    y, f = p >> 1, p & 1
                    hhi = h0hi if y == 0 else h1hi
                    for k in range(nbp2[l]):
                        idxh[rr, off[l] + p * nbp2[l] + k] = (
                            base + ((k ^ hhi) << 1) + f) if k < nb[l] else base
                hc[rr, l, 0] = 0 * nbp2[l] * 128 + h0lo
                hc[rr, l, 1] = 1 * nbp2[l] * 128 + h0lo
                hc[rr, l, 2] = 2 * nbp2[l] * 128 + h1lo
                hc[rr, l, 3] = 3 * nbp2[l] * 128 + h1lo
                for j in range(4 * nbp2[l], seg[l]):
                    idxh[rr, off[l] + j] = base
    return (gridw, nbp2, seg, off, perrow, res,
            idxh.reshape(-1), hc.reshape(-1), fy.reshape(-1))


def _sc_encode(t128, idxh, hch, fyh, seg, off, perrow, res):
    info = plsc.get_sparse_core_info()
    nc = info.num_cores

    def body(t_hbm, idx_hbm, hc_hbm, fy_hbm, enc_hbm,
             fyv, hcv, encv, idxv, slabs, idx_sem, gat_sems):
        wid = lax.axis_index("s") * nc + lax.axis_index("c")
        row_base = wid * ROWS_PER_TEC

        pltpu.sync_copy(fy_hbm.at[pl.ds(row_base * LVL, ROWS_PER_TEC * LVL)], fyv)
        pltpu.sync_copy(hc_hbm.at[pl.ds(row_base * LVL * 4, ROWS_PER_TEC * LVL * 4)], hcv)

        zero16 = jnp.zeros((16,), jnp.int32)
        lane16 = jnp.arange(16, dtype=jnp.int32)

        def row_body(rl, carry):
            row = row_base + rl
            idx_cp = pltpu.make_async_copy(
                idx_hbm.at[pl.ds(row * perrow, perrow)], idxv, idx_sem)
            idx_cp.start()
            idx_cp.wait()
            gat = []
            for l in range(LVL):
                cp = pltpu.make_async_copy(
                    t_hbm.at[idxv.at[pl.ds(off[l], seg[l])]],
                    slabs[l], gat_sems[l])
                cp.start()
                gat.append(cp)

            def hcsplat(l, p):
                return plsc.load_gather(
                    hcv, [jnp.full((16,), (rl * LVL + l) * 4 + p, jnp.int32)])

            for l in range(LVL):
                gat[l].wait()
                slab = slabs[l]
                if l == LVL - 1:
                    h0 = hcsplat(l, 0)
                    h1 = hcsplat(l, 1)

                    def cbody15(ci, c, slab=slab, h0=h0, h1=h1, l=l):
                        basec = ci * 16
                        vcol2 = (lane16 + basec) * 2
                        f0 = plsc.load_gather(slab, [zero16, vcol2 ^ h0])
                        f1 = plsc.load_gather(slab, [zero16, vcol2 ^ h1])
                        encv[2 * l, pl.ds(basec, 16)] = f0
                        encv[2 * l + 1, pl.ds(basec, 16)] = f1
                        return c
                    lax.fori_loop(0, 64, cbody15, 0, unroll=2)
                else:
                    rinv = float(res[l]) / 1024.0
                    vfy = plsc.load_gather(
                        fyv, [jnp.full((16,), rl * LVL + l, jnp.int32)])
                    h00 = hcsplat(l, 0)
                    h01 = hcsplat(l, 1)
                    h10 = hcsplat(l, 2)
                    h11 = hcsplat(l, 3)

                    def cbody(ci, c, rinv=rinv, slab=slab, vfy=vfy,
                              h00=h00, h01=h01, h10=h10, h11=h11, l=l):
                        basec = ci * 16
                        vcolf = (lane16 + basec).astype(jnp.float32)
                        vs = vcolf * jnp.float32(rinv)
                        vix = vs.astype(jnp.int32)
                        vfx = vs - vix.astype(jnp.float32)
                        vix1 = vix + 1
                        c00f0 = plsc.load_gather(slab, [zero16, vix ^ h00])
                        c00f1 = plsc.load_gather(slab, [zero16, vix ^ h01])
                        c10f0 = plsc.load_gather(slab, [zero16, vix1 ^ h00])
                        c10f1 = plsc.load_gather(slab, [zero16, vix1 ^ h01])
                        c01f0 = plsc.load_gather(slab, [zero16, vix ^ h10])
                        c01f1 = plsc.load_gather(slab, [zero16, vix ^ h11])
                        c11f0 = plsc.load_gather(slab, [zero16, vix1 ^ h10])
                        c11f1 = plsc.load_gather(slab, [zero16, vix1 ^ h11])
                        a0 = c00f0 + vfx * (c10f0 - c00f0)
                        a1 = c00f1 + vfx * (c10f1 - c00f1)
                        bb0 = c01f0 + vfx * (c11f0 - c01f0)
                        bb1 = c01f1 + vfx * (c11f1 - c01f1)
                        f0 = a0 + vfy * (bb0 - a0)
                        f1 = a1 + vfy * (bb1 - a1)
                        encv[2 * l, pl.ds(basec, 16)] = f0
                        encv[2 * l + 1, pl.ds(basec, 16)] = f1
                        return c
                    lax.fori_loop(0, 64, cbody, 0, unroll=2)
            pltpu.sync_copy(encv, enc_hbm.at[:, pl.ds(row * 1024, 1024)])
            return carry

        lax.fori_loop(0, ROWS_PER_TEC, row_body, 0)

    mesh = plsc.VectorSubcoreMesh(core_axis_name="c", subcore_axis_name="s")
    scratch = [
        pltpu.VMEM((ROWS_PER_TEC * LVL,), jnp.float32),
        pltpu.VMEM((ROWS_PER_TEC * LVL * 4,), jnp.int32),
        pltpu.VMEM((2 * LVL, 1024), jnp.float32),
        pltpu.VMEM((perrow,), jnp.int32),
        [pltpu.VMEM((seg[l], 128), jnp.float32) for l in range(LVL)],
        pltpu.SemaphoreType.DMA,
        [pltpu.SemaphoreType.DMA for _ in range(LVL)],
    ]
    k = pl.kernel(
        body,
        out_type=jax.ShapeDtypeStruct((2 * LVL, N_PIX), jnp.float32),
        mesh=mesh,
        scratch_types=scratch,
        compiler_params=pltpu.CompilerParams(use_tc_tiling_on_sc=False,
                                             needs_layout_passes=False),
    )
    return k(t128, idxh, hch, fyh)


def _mlp_body(e_ref, w1_ref, b1_ref, w2_ref, b2_ref, w3_ref, b3_ref, o_ref):
    e = e_ref[...]
    h = jnp.dot(w1_ref[...], e, preferred_element_type=jnp.float32) + b1_ref[...]
    h = jnp.maximum(h, 0.0)
    h = jnp.dot(w2_ref[...], h, preferred_element_type=jnp.float32) + b2_ref[...]
    h = jnp.maximum(h, 0.0)
    o = jnp.dot(w3_ref[...], h, preferred_element_type=jnp.float32) + b3_ref[...]
    o_ref[...] = jax.nn.sigmoid(o)


def _mlp(enc, w1t, b1, w2t, b2, w3t, b3):
    bn = 4096
    grid = (N_PIX // bn,)
    out = pl.pallas_call(
        _mlp_body,
        grid=grid,
        in_specs=[
            pl.BlockSpec((2 * LVL, bn), lambda i: (0, i)),
            pl.BlockSpec((HIDDEN, 2 * LVL), lambda i: (0, 0)),
            pl.BlockSpec((HIDDEN, 1), lambda i: (0, 0)),
            pl.BlockSpec((HIDDEN, HIDDEN), lambda i: (0, 0)),
            pl.BlockSpec((HIDDEN, 1), lambda i: (0, 0)),
            pl.BlockSpec((8, HIDDEN), lambda i: (0, 0)),
            pl.BlockSpec((8, 1), lambda i: (0, 0)),
        ],
        out_specs=pl.BlockSpec((8, bn), lambda i: (0, i)),
        out_shape=jax.ShapeDtypeStruct((8, N_PIX), jnp.float32),
    )(enc, w1t, b1, w2t, b2, w3t, b3)
    return out[:3]


def kernel(tables, W1, b1, W2, b2, W3, b3):
    (gridw, nbp2, seg, off, perrow, res,
     idx_np, hc_np, fy_np) = _host_consts()
    t128 = tables.reshape(LVL, NSPAN, 128, F_DIM)
    t128 = t128.transpose(0, 1, 3, 2).reshape(LVL * ROWS_PER_LVL, 128)
    enc = _sc_encode(
        t128,
        jnp.asarray(idx_np),
        jnp.asarray(hc_np),
        jnp.asarray(fy_np),
        seg, off, perrow, res)
    w1t = W1.T
    w2t = W2.T
    w3t = jnp.zeros((8, HIDDEN), jnp.float32).at[:3].set(W3.T)
    b3p = jnp.zeros((8, 1), jnp.float32).at[:3, 0].set(b3)
    out = _mlp(enc, w1t, b1.reshape(HIDDEN, 1), w2t, b2.reshape(HIDDEN, 1),
               w3t, b3p)
    return out.reshape(3, H_RES, W_RES)[None]


# speedup vs baseline: 319.0211x; 2.6704x over previous
import functools

import numpy as np
import jax
import jax.numpy as jnp
from jax import lax
from jax.experimental import pallas as pl
from jax.experimental.pallas import tpu as pltpu
from jax.experimental.pallas import tpu_sc as plsc

W_RES = 1024
H_RES = 1024
LVL = 16
F_DIM = 2
LOG2_T = 19
TBL = 2 ** LOG2_T
HASH_K = np.uint32(2654435761)
HMASK = np.uint32(TBL - 1)
HIDDEN = 64
N_PIX = W_RES * H_RES

ROWS_PER_TEC = H_RES // 32
NSPAN = TBL // 128
ROWS_PER_LVL = 2 * NSPAN


def _ceil8(n):
    return (n + 7) // 8 * 8


def _next_pow2(n):
    p = 1
    while p < n:
        p *= 2
    return p


@functools.lru_cache(maxsize=1)
def _host_consts():
    b = np.exp((np.log(2048.0) - np.log(16.0)) / (LVL - 1))
    res = np.floor(16.0 * (b ** np.arange(LVL))).astype(np.float32)
    norm = (np.arange(1024, dtype=np.float32) / np.float32(1024))

    gridw = []
    for l in range(LVL):
        r = np.float32(res[l])
        sx = (norm * r).astype(np.float32)
        ix = np.floor(sx).astype(np.int32)
        gridw.append(int(ix.max()) + 2)

    nb = [-(-gridw[l] // 128) for l in range(LVL)]
    nbp2 = [_next_pow2(nb[l]) for l in range(LVL)]
    seg = [_ceil8(4 * nbp2[l]) for l in range(LVL - 1)] + [2 * nbp2[LVL - 1]]
    off = np.cumsum([0] + seg).tolist()
    perrow = off[-1]

    fy = np.zeros((1024, LVL), np.float32)
    hc = np.zeros((1024, LVL, 4), np.int32)
    idxh = np.zeros((1024, perrow), np.int32)
    for l in range(LVL):
        r = np.float32(res[l])
        sy = (norm * r).astype(np.float32)
        py = np.floor(sy)
        iy = py.astype(np.uint32)
        fy[:, l] = sy - py
        hy0 = ((iy * HASH_K) & HMASK).astype(np.int64)
        hy1 = (((iy + np.uint32(1)) * HASH_K) & HMASK).astype(np.int64)
        base = l * ROWS_PER_LVL
        for rr in range(1024):
            h0hi, h0lo = int(hy0[rr]) >> 7, int(hy0[rr]) & 127
            h1hi, h1lo = int(hy1[rr]) >> 7, int(hy1[rr]) & 127
            if l == LVL - 1:
                for f in range(2):
                    for k in range(nbp2[l]):
                        idxh[rr, off[l] + f * nbp2[l] + k] = (
                            base + ((k ^ h0hi) << 1) + f) if k < nb[l] else base
                hc[rr, l, 0] = h0lo
                hc[rr, l, 1] = nbp2[l] * 128 + h0lo
            else:
                for p in range(4):
                    y, f = p >> 1, p & 1
                    hhi = h0hi if y == 0 else h1hi
                    for k in range(nbp2[l]):
                        idxh[rr, off[l] + p * nbp2[l] + k] = (
                            base + ((k ^ hhi) << 1) + f) if k < nb[l] else base
                hc[rr, l, 0] = 0 * nbp2[l] * 128 + h0lo
                hc[rr, l, 1] = 1 * nbp2[l] * 128 + h0lo
                hc[rr, l, 2] = 2 * nbp2[l] * 128 + h1lo
                hc[rr, l, 3] = 3 * nbp2[l] * 128 + h1lo
                for j in range(4 * nbp2[l], seg[l]):
                    idxh[rr, off[l] + j] = base
    return (gridw, nbp2, seg, off, perrow, res,
            idxh.reshape(-1), hc.reshape(-1), fy.reshape(-1))


def _sc_encode(t128, idxh, hch, fyh, seg, off, perrow, res):
    info = plsc.get_sparse_core_info()
    nc = info.num_cores

    def body(t_hbm, idx_hbm, hc_hbm, fy_hbm, enc_hbm,
             fyv, hcv, encv, idxv, slabs, idx_sem, gat_sems):
        wid = lax.axis_index("s") * nc + lax.axis_index("c")
        row_base = wid * ROWS_PER_TEC

        pltpu.sync_copy(fy_hbm.at[pl.ds(row_base * LVL, ROWS_PER_TEC * LVL)], fyv)
        pltpu.sync_copy(hc_hbm.at[pl.ds(row_base * LVL * 4, ROWS_PER_TEC * LVL * 4)], hcv)

        zero16 = jnp.zeros((16,), jnp.int32)
        lane16 = jnp.arange(16, dtype=jnp.int32)

        def row_body(rl, carry):
            row = row_base + rl
            idx_cp = pltpu.make_async_copy(
                idx_hbm.at[pl.ds(row * perrow, perrow)], idxv, idx_sem)
            idx_cp.start()
            idx_cp.wait()
            gat = []
            for l in range(LVL):
                cp = pltpu.make_async_copy(
                    t_hbm.at[idxv.at[pl.ds(off[l], seg[l])]],
                    slabs[l], gat_sems[l])
                cp.start()
                gat.append(cp)

            def hcsplat(l, p):
                return plsc.load_gather(
                    hcv, [jnp.full((16,), (rl * LVL + l) * 4 + p, jnp.int32)])

            for l in range(LVL):
                gat[l].wait()
                slab = slabs[l]
                fbase0 = (2 * l) * 128
                fbase1 = (2 * l + 1) * 128
                if l == LVL - 1:
                    h0 = hcsplat(l, 0)
                    h1 = hcsplat(l, 1)

                    @plsc.parallel_loop(0, 64, unroll=4)
                    def cbody15(ci, slab=slab, h0=h0, h1=h1,
                                fbase0=fbase0, fbase1=fbase1):
                        basec = ci * 16
                        eoff = (ci // 8) * 4096 + (ci % 8) * 16
                        vcol2 = (lane16 + basec) * 2
                        f0 = plsc.load_gather(slab, [zero16, vcol2 ^ h0])
                        f1 = plsc.load_gather(slab, [zero16, vcol2 ^ h1])
                        encv[pl.ds(eoff + fbase0, 16)] = f0
                        encv[pl.ds(eoff + fbase1, 16)] = f1
                else:
                    rinv = float(res[l]) / 1024.0
                    vfy = plsc.load_gather(
                        fyv, [jnp.full((16,), rl * LVL + l, jnp.int32)])
                    h00 = hcsplat(l, 0)
                    h01 = hcsplat(l, 1)
                    h10 = hcsplat(l, 2)
                    h11 = hcsplat(l, 3)

                    @plsc.parallel_loop(0, 64, unroll=4)
                    def cbody(ci, rinv=rinv, slab=slab, vfy=vfy,
                              h00=h00, h01=h01, h10=h10, h11=h11,
                              fbase0=fbase0, fbase1=fbase1):
                        basec = ci * 16
                        eoff = (ci // 8) * 4096 + (ci % 8) * 16
                        vcolf = (lane16 + basec).astype(jnp.float32)
                        vs = vcolf * jnp.float32(rinv)
                        vix = vs.astype(jnp.int32)
                        vfx = vs - vix.astype(jnp.float32)
                        vix1 = vix + 1
                        c00f0 = plsc.load_gather(slab, [zero16, vix ^ h00])
                        c00f1 = plsc.load_gather(slab, [zero16, vix ^ h01])
                        c10f0 = plsc.load_gather(slab, [zero16, vix1 ^ h00])
                        c10f1 = plsc.load_gather(slab, [zero16, vix1 ^ h01])
                        c01f0 = plsc.load_gather(slab, [zero16, vix ^ h10])
                        c01f1 = plsc.load_gather(slab, [zero16, vix ^ h11])
                        c11f0 = plsc.load_gather(slab, [zero16, vix1 ^ h10])
                        c11f1 = plsc.load_gather(slab, [zero16, vix1 ^ h11])
                        a0 = c00f0 + vfx * (c10f0 - c00f0)
                        a1 = c00f1 + vfx * (c10f1 - c00f1)
                        bb0 = c01f0 + vfx * (c11f0 - c01f0)
                        bb1 = c01f1 + vfx * (c11f1 - c01f1)
                        f0 = a0 + vfy * (bb0 - a0)
                        f1 = a1 + vfy * (bb1 - a1)
                        encv[pl.ds(eoff + fbase0, 16)] = f0
                        encv[pl.ds(eoff + fbase1, 16)] = f1
            pltpu.sync_copy(encv, enc_hbm.at[pl.ds(row * 32768, 32768)])
            return carry

        lax.fori_loop(0, ROWS_PER_TEC, row_body, 0)

    mesh = plsc.VectorSubcoreMesh(core_axis_name="c", subcore_axis_name="s")
    scratch = [
        pltpu.VMEM((ROWS_PER_TEC * LVL,), jnp.float32),
        pltpu.VMEM((ROWS_PER_TEC * LVL * 4,), jnp.int32),
        pltpu.VMEM((2 * LVL * 1024,), jnp.float32),
        pltpu.VMEM((perrow,), jnp.int32),
        [pltpu.VMEM((seg[l], 128), jnp.float32) for l in range(LVL)],
        pltpu.SemaphoreType.DMA,
        [pltpu.SemaphoreType.DMA for _ in range(LVL)],
    ]
    k = pl.kernel(
        body,
        out_type=jax.ShapeDtypeStruct((2 * LVL * N_PIX,), jnp.float32),
        mesh=mesh,
        scratch_types=scratch,
        compiler_params=pltpu.CompilerParams(use_tc_tiling_on_sc=False,
                                             needs_layout_passes=False),
    )
    return k(t128, idxh, hch, fyh)


def _mlp_body(e_ref, w1_ref, b1_ref, w2_ref, b2_ref, w3_ref, b3_ref, o_ref):
    e3 = e_ref[...]
    bp = e3.shape[0]
    e = jnp.transpose(e3, (1, 0, 2)).reshape(2 * LVL, bp * 128)
    h = jnp.dot(w1_ref[...], e, preferred_element_type=jnp.float32) + b1_ref[...]
    h = jnp.maximum(h, 0.0)
    h = jnp.dot(w2_ref[...], h, preferred_element_type=jnp.float32) + b2_ref[...]
    h = jnp.maximum(h, 0.0)
    o = jnp.dot(w3_ref[...], h, preferred_element_type=jnp.float32) + b3_ref[...]
    o_ref[...] = jax.nn.sigmoid(o)


def _mlp(enc3, w1t, b1, w2t, b2, w3t, b3):
    bp = 32
    grid = (N_PIX // (128 * bp),)
    out = pl.pallas_call(
        _mlp_body,
        grid=grid,
        in_specs=[
            pl.BlockSpec((bp, 2 * LVL, 128), lambda i: (i, 0, 0)),
            pl.BlockSpec((HIDDEN, 2 * LVL), lambda i: (0, 0)),
            pl.BlockSpec((HIDDEN, 1), lambda i: (0, 0)),
            pl.BlockSpec((HIDDEN, HIDDEN), lambda i: (0, 0)),
            pl.BlockSpec((HIDDEN, 1), lambda i: (0, 0)),
            pl.BlockSpec((8, HIDDEN), lambda i: (0, 0)),
            pl.BlockSpec((8, 1), lambda i: (0, 0)),
        ],
        out_specs=pl.BlockSpec((8, bp * 128), lambda i: (0, i)),
        out_shape=jax.ShapeDtypeStruct((8, N_PIX), jnp.float32),
    )(enc3, w1t, b1, w2t, b2, w3t, b3)
    return out[:3]


def kernel(tables, W1, b1, W2, b2, W3, b3):
    (gridw, nbp2, seg, off, perrow, res,
     idx_np, hc_np, fy_np) = _host_consts()
    t128 = tables.reshape(LVL, NSPAN, 128, F_DIM)
    t128 = t128.transpose(0, 1, 3, 2).reshape(LVL * ROWS_PER_LVL, 128)
    enc = _sc_encode(
        t128,
        jnp.asarray(idx_np),
        jnp.asarray(hc_np),
        jnp.asarray(fy_np),
        seg, off, perrow, res)
    enc3 = enc.reshape(N_PIX // 128, 2 * LVL, 128)
    w1t = W1.T
    w2t = W2.T
    w3t = jnp.zeros((8, HIDDEN), jnp.float32).at[:3].set(W3.T)
    b3p = jnp.zeros((8, 1), jnp.float32).at[:3, 0].set(b3)
    out = _mlp(enc3, w1t, b1.reshape(HIDDEN, 1), w2t, b2.reshape(HIDDEN, 1),
               w3t, b3p)
    return out.reshape(3, H_RES, W_RES)[None]
